# baseline (speedup 1.0000x reference)
import jax
import jax.numpy as jnp
from jax import lax
from jax.experimental import pallas as pl

EMB_D = 64
USR_D = 16
NEXP = 64
PACK = 8
ROWS = 512
W8 = PACK * NEXP


def _segmax(x):
    m = jnp.max(x.reshape(ROWS, PACK, NEXP), axis=2, keepdims=True)
    return jnp.broadcast_to(m, (ROWS, PACK, NEXP)).reshape(ROWS, W8)


def _gate_body(h_ref, u_ref, wg_ref, bg_ref, wb_ref, bb_ref, wl_ref,
               bl_ref, out_ref):
    u = u_ref[...]
    h = h_ref[...]
    gamma = jnp.dot(u, wg_ref[...], preferred_element_type=jnp.float32)
    gamma = gamma + bg_ref[...]
    beta = jnp.dot(u, wb_ref[...], preferred_element_type=jnp.float32)
    beta = beta + bb_ref[...]
    h_t = h * (1.0 + gamma) + beta
    logits = jnp.dot(h_t, wl_ref[...], preferred_element_type=jnp.float32)
    logits = logits + bl_ref[...]

    row = lax.broadcasted_iota(jnp.int32, (W8, W8), 0)
    col = lax.broadcasted_iota(jnp.int32, (W8, W8), 1)
    lt = ((row <= col) &
          (lax.shift_right_logical(row, 6) ==
           lax.shift_right_logical(col, 6))).astype(jnp.float32)

    m1 = _segmax(logits)
    eq1 = logits == m1
    cs1 = jnp.dot(eq1.astype(jnp.float32), lt,
                  preferred_element_type=jnp.float32)
    mask1 = eq1 & (cs1 == 1.0)
    l2 = jnp.where(mask1, -jnp.inf, logits)
    m2 = _segmax(l2)
    eq2 = l2 == m2
    cs2 = jnp.dot(eq2.astype(jnp.float32), lt,
                  preferred_element_type=jnp.float32)
    mask2 = eq2 & (cs2 == 1.0)

    e = jnp.exp(m2 - m1)
    p1 = 1.0 / (1.0 + e)
    p2 = 1.0 - p1
    out_ref[...] = jnp.where(mask1, p1, jnp.where(mask2, p2, 0.0))


def kernel(h, u, Wg, bg, Wb, bb, Wl, bl):
    n = h.shape[0]
    h8 = h.reshape(n // PACK, PACK * EMB_D)
    u8 = u.reshape(n // PACK, PACK * USR_D)
    eye = jnp.eye(PACK, dtype=jnp.float32)
    wg8 = jnp.kron(eye, Wg.T)
    wb8 = jnp.kron(eye, Wb.T)
    wl8 = jnp.kron(eye, Wl.T)
    bg8 = jnp.tile(bg, PACK)[None, :]
    bb8 = jnp.tile(bb, PACK)[None, :]
    bl8 = jnp.tile(bl, PACK)[None, :]
    grid = (n // PACK // ROWS,)
    w8 = pl.pallas_call(
        _gate_body,
        grid=grid,
        in_specs=[
            pl.BlockSpec((ROWS, PACK * EMB_D), lambda i: (i, 0)),
            pl.BlockSpec((ROWS, PACK * USR_D), lambda i: (i, 0)),
            pl.BlockSpec((PACK * USR_D, W8), lambda i: (0, 0)),
            pl.BlockSpec((1, W8), lambda i: (0, 0)),
            pl.BlockSpec((PACK * USR_D, W8), lambda i: (0, 0)),
            pl.BlockSpec((1, W8), lambda i: (0, 0)),
            pl.BlockSpec((PACK * EMB_D, W8), lambda i: (0, 0)),
            pl.BlockSpec((1, W8), lambda i: (0, 0)),
        ],
        out_specs=pl.BlockSpec((ROWS, W8), lambda i: (i, 0)),
        out_shape=jax.ShapeDtypeStruct((n // PACK, W8), jnp.float32),
    )(h8, u8, wg8, bg8, wb8, bb8, wl8, bl8)
    return w8.reshape(n, NEXP)
